# baseline (speedup 1.0000x reference)
import jax
import jax.numpy as jnp
from jax import lax
from jax.experimental import pallas as pl
from jax.experimental.pallas import tpu as pltpu
from jax.experimental.pallas import tpu_sc as plsc

B = 8
C_IN = 96
C_OUT = 192
H = 224
W = 224
PLANE = H * W
N_PLANES = B * C_IN
NW = 32
PLANES_PER_W = N_PLANES // NW

_mesh = plsc.VectorSubcoreMesh(core_axis_name="c", subcore_axis_name="s")


def _body(x_hbm, out_hbm, zbuf, sem):
    cid = lax.axis_index("c")
    sid = lax.axis_index("s")
    wid = sid * 2 + cid

    zeros16 = jnp.zeros((16,), jnp.float32)

    def zinit(i, carry):
        zbuf[pl.ds(i * 16, 16)] = zeros16
        return carry

    lax.fori_loop(0, PLANE // 16, zinit, 0)

    base = wid * PLANES_PER_W
    handles = []
    for k in range(PLANES_PER_W):
        p = base + k
        handles.append(pltpu.async_copy(x_hbm.at[p], out_hbm.at[2 * p], sem))
        handles.append(pltpu.async_copy(zbuf, out_hbm.at[2 * p + 1], sem))
    for h in handles:
        h.wait()


def kernel(x):
    xr = x.reshape(N_PLANES, PLANE)
    fn = pl.kernel(
        _body,
        out_type=jax.ShapeDtypeStruct((2 * N_PLANES, PLANE), jnp.float32),
        mesh=_mesh,
        scratch_types=[
            pltpu.VMEM((PLANE,), jnp.float32),
            pltpu.SemaphoreType.DMA,
        ],
    )
    out = fn(xr)
    return out.reshape(B, C_OUT, H, W)


# speedup vs baseline: 7.6170x; 7.6170x over previous
import jax
import jax.numpy as jnp
from jax import lax
from jax.experimental import pallas as pl
from jax.experimental.pallas import tpu as pltpu
from jax.experimental.pallas import tpu_sc as plsc

B = 8
C_IN = 96
C_OUT = 192
H = 224
W = 224
PLANE = H * W
CH = PLANE // 2
N_CHUNKS = B * C_IN * 2
NW = 32
CPW = N_CHUNKS // NW
NBUF = 4

_mesh = plsc.VectorSubcoreMesh(core_axis_name="c", subcore_axis_name="s")


def _body(x_hbm, out_hbm, b0, b1, b2, b3, zbuf,
          si0, si1, si2, si3, so0, so1, so2, so3, sz):
    cid = lax.axis_index("c")
    sid = lax.axis_index("s")
    wid = sid * 2 + cid

    bufs = [b0, b1, b2, b3]
    sem_in = [si0, si1, si2, si3]
    sem_out = [so0, so1, so2, so3]

    zeros16 = jnp.zeros((16,), jnp.float32)

    def zinit(i, carry):
        zbuf[pl.ds(i * 16, 16)] = zeros16
        return carry

    lax.fori_loop(0, CH // 16, zinit, 0)

    cbase = wid * CPW
    obase = 2 * cbase

    inh = [None] * CPW
    outh = [None] * CPW
    zh = []
    for j in range(CPW + 1):
        if j < CPW:
            b = j % NBUF
            if j >= NBUF:
                outh[j - NBUF].wait()
            inh[j] = pltpu.async_copy(x_hbm.at[cbase + j], bufs[b], sem_in[b])
            zrow = obase + 4 * (j // 2) + 2 + (j % 2)
            zh.append(pltpu.async_copy(zbuf, out_hbm.at[zrow], sz))
        if j >= 1:
            jj = j - 1
            bb = jj % NBUF
            inh[jj].wait()
            orow = obase + 4 * (jj // 2) + (jj % 2)
            outh[jj] = pltpu.async_copy(bufs[bb], out_hbm.at[orow], sem_out[bb])
    for j in range(CPW - NBUF, CPW):
        outh[j].wait()
    for h in zh:
        h.wait()


def kernel(x):
    xr = x.reshape(N_CHUNKS, CH)
    fn = pl.kernel(
        _body,
        out_type=jax.ShapeDtypeStruct((2 * N_CHUNKS, CH), jnp.float32),
        mesh=_mesh,
        scratch_types=(
            [pltpu.VMEM((CH,), jnp.float32) for _ in range(NBUF + 1)]
            + [pltpu.SemaphoreType.DMA for _ in range(2 * NBUF + 1)]
        ),
    )
    out = fn(xr)
    return out.reshape(B, C_OUT, H, W)


# speedup vs baseline: 24.6867x; 3.2410x over previous
import jax
import jax.numpy as jnp
from jax import lax
from jax.experimental import pallas as pl
from jax.experimental.pallas import tpu as pltpu
from jax.experimental.pallas import tpu_sc as plsc

B = 8
C_IN = 96
C_OUT = 192
H = 224
W = 224
HH = H // 2
N_PLANES = B * C_IN
NW = 32
PPW = N_PLANES // NW
CPW = PPW * 2
NBUF = 3

_mesh = plsc.VectorSubcoreMesh(core_axis_name="c", subcore_axis_name="s")


def _body(x_hbm, out_hbm, b0, b1, b2, zbuf,
          si0, si1, si2, so0, so1, so2, sz):
    cid = lax.axis_index("c")
    sid = lax.axis_index("s")
    wid = sid * 2 + cid

    bufs = [b0, b1, b2]
    sem_in = [si0, si1, si2]
    sem_out = [so0, so1, so2]

    zeros16 = jnp.zeros((16,), jnp.float32)

    def zrow(i, carry):
        def zcol(j, carry2):
            zbuf[i, pl.ds(j * 16, 16)] = zeros16
            return carry2

        return lax.fori_loop(0, W // 16, zcol, carry)

    lax.fori_loop(0, HH, zrow, 0)

    pbase = wid * PPW
    inh = [None] * CPW
    outh = [None] * CPW
    zh = []
    for j in range(CPW + 1):
        if j < CPW:
            b = j % NBUF
            if j >= NBUF:
                outh[j - NBUF].wait()
            p = pbase + j // 2
            r0 = (j % 2) * HH
            inh[j] = pltpu.async_copy(
                x_hbm.at[p, pl.ds(r0, HH)], bufs[b], sem_in[b])
            zh.append(pltpu.async_copy(
                zbuf, out_hbm.at[2 * p + 1, pl.ds(r0, HH)], sz))
        if j >= 1:
            jj = j - 1
            bb = jj % NBUF
            inh[jj].wait()
            pp = pbase + jj // 2
            rr0 = (jj % 2) * HH
            outh[jj] = pltpu.async_copy(
                bufs[bb], out_hbm.at[2 * pp, pl.ds(rr0, HH)], sem_out[bb])
    for j in range(CPW - NBUF, CPW):
        outh[j].wait()
    for h in zh:
        h.wait()


def kernel(x):
    xr = x.reshape(N_PLANES, H, W)
    fn = pl.kernel(
        _body,
        out_type=jax.ShapeDtypeStruct((2 * N_PLANES, H, W), jnp.float32),
        mesh=_mesh,
        scratch_types=(
            [pltpu.VMEM((HH, W), jnp.float32) for _ in range(NBUF + 1)]
            + [pltpu.SemaphoreType.DMA for _ in range(2 * NBUF + 1)]
        ),
    )
    out = fn(xr)
    return out.reshape(B, C_OUT, H, W)


# speedup vs baseline: 24.9412x; 1.0103x over previous
import jax
import jax.numpy as jnp
from jax import lax
from jax.experimental import pallas as pl
from jax.experimental.pallas import tpu as pltpu
from jax.experimental.pallas import tpu_sc as plsc

B = 8
C_IN = 96
C_OUT = 192
H = 224
W = 224
HH = H // 2
N_PLANES = B * C_IN
NW = 32
NS = 16
PPW = N_PLANES // NW
CPW = PPW * 2
NBUF = 4
ZR = 8

_mesh = plsc.VectorSubcoreMesh(core_axis_name="c", subcore_axis_name="s")


def _body(x_hbm, out_hbm, b0, b1, b2, b3, zseed, zshared,
          si0, si1, si2, si3, so0, so1, so2, so3, sz):
    cid = lax.axis_index("c")
    sid = lax.axis_index("s")
    wid = sid * 2 + cid

    bufs = [b0, b1, b2, b3]
    sem_in = [si0, si1, si2, si3]
    sem_out = [so0, so1, so2, so3]

    pbase = wid * PPW
    inh = [None] * CPW

    for j in range(NBUF):
        p = pbase + j // 2
        r0 = (j % 2) * HH
        inh[j] = pltpu.async_copy(
            x_hbm.at[p, pl.ds(r0, HH)], bufs[j], sem_in[j])

    zeros16 = jnp.zeros((16,), jnp.float32)

    def zrow(i, carry):
        def zcol(j, carry2):
            zseed[i, pl.ds(j * 16, 16)] = zeros16
            return carry2

        return lax.fori_loop(0, W // 16, zcol, carry)

    lax.fori_loop(0, ZR, zrow, 0)
    pltpu.sync_copy(zseed, zshared.at[pl.ds(sid * ZR, ZR)])
    plsc.subcore_barrier()
    zsrc = zshared.at[pl.ds(0, HH)]

    outh = [None] * CPW
    zh = []
    for j in range(CPW + 1):
        if j < CPW:
            b = j % NBUF
            if j >= NBUF:
                outh[j - NBUF].wait()
                p = pbase + j // 2
                r0 = (j % 2) * HH
                inh[j] = pltpu.async_copy(
                    x_hbm.at[p, pl.ds(r0, HH)], bufs[b], sem_in[b])
            p = pbase + j // 2
            r0 = (j % 2) * HH
            zh.append(pltpu.async_copy(
                zsrc, out_hbm.at[2 * p + 1, pl.ds(r0, HH)], sz))
        if j >= 1:
            jj = j - 1
            bb = jj % NBUF
            inh[jj].wait()
            pp = pbase + jj // 2
            rr0 = (jj % 2) * HH
            outh[jj] = pltpu.async_copy(
                bufs[bb], out_hbm.at[2 * pp, pl.ds(rr0, HH)], sem_out[bb])
    for j in range(CPW - NBUF, CPW):
        outh[j].wait()
    for h in zh:
        h.wait()


def kernel(x):
    xr = x.reshape(N_PLANES, H, W)
    fn = pl.kernel(
        _body,
        out_type=jax.ShapeDtypeStruct((2 * N_PLANES, H, W), jnp.float32),
        mesh=_mesh,
        scratch_types=(
            [pltpu.VMEM((HH, W), jnp.float32) for _ in range(NBUF)]
            + [pltpu.VMEM((ZR, W), jnp.float32)]
            + [pltpu.VMEM_SHARED((NS * ZR, W), jnp.float32)]
            + [pltpu.SemaphoreType.DMA for _ in range(2 * NBUF + 1)]
        ),
    )
    out = fn(xr)
    return out.reshape(B, C_OUT, H, W)


# speedup vs baseline: 25.1426x; 1.0081x over previous
import jax
import jax.numpy as jnp
from jax import lax
from jax.experimental import pallas as pl
from jax.experimental.pallas import tpu as pltpu
from jax.experimental.pallas import tpu_sc as plsc

B = 8
C_IN = 96
C_OUT = 192
H = 224
W = 224
N_PLANES = B * C_IN
NW = 32
NS = 16
PPW = N_PLANES // NW
NBUF = 2
ZR = 16

_mesh = plsc.VectorSubcoreMesh(core_axis_name="c", subcore_axis_name="s")


def _body(x_hbm, out_hbm, b0, b1, zseed, zshared, si0, si1, so0, so1, sz):
    cid = lax.axis_index("c")
    sid = lax.axis_index("s")
    wid = sid * 2 + cid

    bufs = [b0, b1]
    sem_in = [si0, si1]
    sem_out = [so0, so1]

    pbase = wid * PPW
    inh = [None] * PPW

    for j in range(NBUF):
        inh[j] = pltpu.async_copy(x_hbm.at[pbase + j], bufs[j], sem_in[j])

    zeros16 = jnp.zeros((16,), jnp.float32)

    def zrow(i, carry):
        def zcol(j, carry2):
            zseed[i, pl.ds(j * 16, 16)] = zeros16
            return carry2

        return lax.fori_loop(0, W // 16, zcol, carry)

    lax.fori_loop(0, ZR, zrow, 0)
    pltpu.sync_copy(zseed, zshared.at[pl.ds(sid * ZR, ZR)])
    plsc.subcore_barrier()
    zsrc = zshared.at[pl.ds(0, H)]

    outh = [None] * PPW
    zh = []
    for j in range(PPW + 1):
        if j < PPW:
            b = j % NBUF
            if j >= NBUF:
                outh[j - NBUF].wait()
                inh[j] = pltpu.async_copy(
                    x_hbm.at[pbase + j], bufs[b], sem_in[b])
            zh.append(pltpu.async_copy(
                zsrc, out_hbm.at[2 * (pbase + j) + 1], sz))
        if j >= 1:
            jj = j - 1
            inh[jj].wait()
            outh[jj] = pltpu.async_copy(
                bufs[jj % NBUF], out_hbm.at[2 * (pbase + jj)],
                sem_out[jj % NBUF])
    for j in range(PPW - NBUF, PPW):
        outh[j].wait()
    for h in zh:
        h.wait()


def kernel(x):
    xr = x.reshape(N_PLANES, H, W)
    fn = pl.kernel(
        _body,
        out_type=jax.ShapeDtypeStruct((2 * N_PLANES, H, W), jnp.float32),
        mesh=_mesh,
        scratch_types=(
            [pltpu.VMEM((H, W), jnp.float32) for _ in range(NBUF)]
            + [pltpu.VMEM((ZR, W), jnp.float32)]
            + [pltpu.VMEM_SHARED((NS * ZR, W), jnp.float32)]
            + [pltpu.SemaphoreType.DMA for _ in range(2 * NBUF + 1)]
        ),
    )
    out = fn(xr)
    return out.reshape(B, C_OUT, H, W)


# speedup vs baseline: 25.4016x; 1.0103x over previous
import jax
import jax.numpy as jnp
from jax import lax
from jax.experimental import pallas as pl
from jax.experimental.pallas import tpu as pltpu
from jax.experimental.pallas import tpu_sc as plsc

B = 8
C_IN = 96
C_OUT = 192
H = 224
W = 224
N_PLANES = B * C_IN
NW = 32
NS = 16
PPW = N_PLANES // NW
NBUF = 2
ZR = 16

_mesh = plsc.VectorSubcoreMesh(core_axis_name="c", subcore_axis_name="s")


def _body(x_hbm, out_hbm, b0, b1, zseed, zshared, si0, si1, so0, so1, sz):
    cid = lax.axis_index("c")
    sid = lax.axis_index("s")
    wid = sid * 2 + cid

    bufs = [b0, b1]
    sem_in = [si0, si1]
    sem_out = [so0, so1]

    inh = [None] * PPW

    for j in range(NBUF):
        inh[j] = pltpu.async_copy(x_hbm.at[j * NW + wid], bufs[j], sem_in[j])

    zeros16 = jnp.zeros((16,), jnp.float32)

    def zrow(i, carry):
        def zcol(j, carry2):
            zseed[i, pl.ds(j * 16, 16)] = zeros16
            return carry2

        return lax.fori_loop(0, W // 16, zcol, carry)

    lax.fori_loop(0, ZR, zrow, 0)
    pltpu.sync_copy(zseed, zshared.at[pl.ds(sid * ZR, ZR)])
    plsc.subcore_barrier()
    zsrc = zshared.at[pl.ds(0, H)]

    outh = [None] * PPW
    zh = []
    for j in range(PPW + 1):
        if j < PPW:
            b = j % NBUF
            if j >= NBUF:
                outh[j - NBUF].wait()
                inh[j] = pltpu.async_copy(
                    x_hbm.at[j * NW + wid], bufs[b], sem_in[b])
            zh.append(pltpu.async_copy(
                zsrc, out_hbm.at[2 * (j * NW + wid) + 1], sz))
        if j >= 1:
            jj = j - 1
            inh[jj].wait()
            outh[jj] = pltpu.async_copy(
                bufs[jj % NBUF], out_hbm.at[2 * (jj * NW + wid)],
                sem_out[jj % NBUF])
    for j in range(PPW - NBUF, PPW):
        outh[j].wait()
    for h in zh:
        h.wait()


def kernel(x):
    xr = x.reshape(N_PLANES, H, W)
    fn = pl.kernel(
        _body,
        out_type=jax.ShapeDtypeStruct((2 * N_PLANES, H, W), jnp.float32),
        mesh=_mesh,
        scratch_types=(
            [pltpu.VMEM((H, W), jnp.float32) for _ in range(NBUF)]
            + [pltpu.VMEM((ZR, W), jnp.float32)]
            + [pltpu.VMEM_SHARED((NS * ZR, W), jnp.float32)]
            + [pltpu.SemaphoreType.DMA for _ in range(2 * NBUF + 1)]
        ),
    )
    out = fn(xr)
    return out.reshape(B, C_OUT, H, W)
